# baseline (speedup 1.0000x reference)
import jax
import jax.numpy as jnp
from jax.experimental import pallas as pl
from jax.experimental.pallas import tpu as pltpu

_N = 10000
_BM1 = 2000
_BM2 = 400


def _transform_kernel(ax_ref, w0t_ref, b0_ref, a0_ref, w1t_ref, b1_ref, t_ref):
    x = jnp.dot(ax_ref[...], w0t_ref[...], preferred_element_type=jnp.float32)
    x = x + b0_ref[...]
    a0 = a0_ref[0, 0]
    x = jnp.where(x >= 0, x, a0 * x)
    t = jnp.dot(x, w1t_ref[...], preferred_element_type=jnp.float32)
    t_ref[...] = t + b1_ref[...]


def _aggregate_kernel(a_ref, t_ref, a1_ref, out_ref):
    acc = jnp.dot(a_ref[...], t_ref[...], preferred_element_type=jnp.float32)
    a1 = a1_ref[0, 0]
    out_ref[...] = jnp.where(acc >= 0, acc, a1 * acc)


def kernel(A, AX, W0, b0, a0, W1, b1, a1):
    n, d = AX.shape
    h = W0.shape[0]

    w0t = W0.T
    w1t = W1.T
    b0r = b0.reshape(1, h)
    b1r = b1.reshape(1, h)
    a0r = a0.reshape(1, 1)
    a1r = a1.reshape(1, 1)

    t = pl.pallas_call(
        _transform_kernel,
        grid=(n // _BM1,),
        in_specs=[
            pl.BlockSpec((_BM1, d), lambda i: (i, 0)),
            pl.BlockSpec((d, h), lambda i: (0, 0)),
            pl.BlockSpec((1, h), lambda i: (0, 0)),
            pl.BlockSpec(memory_space=pltpu.SMEM),
            pl.BlockSpec((h, h), lambda i: (0, 0)),
            pl.BlockSpec((1, h), lambda i: (0, 0)),
        ],
        out_specs=pl.BlockSpec((_BM1, h), lambda i: (i, 0)),
        out_shape=jax.ShapeDtypeStruct((n, h), jnp.float32),
    )(AX, w0t, b0r, a0r, w1t, b1r)

    out = pl.pallas_call(
        _aggregate_kernel,
        grid=(n // _BM2,),
        in_specs=[
            pl.BlockSpec((_BM2, n), lambda i: (i, 0)),
            pl.BlockSpec((n, h), lambda i: (0, 0)),
            pl.BlockSpec(memory_space=pltpu.SMEM),
        ],
        out_specs=pl.BlockSpec((_BM2, h), lambda i: (i, 0)),
        out_shape=jax.ShapeDtypeStruct((n, h), jnp.float32),
    )(A, t, a1r)

    return out[None, :, :]


# speedup vs baseline: 1.0594x; 1.0594x over previous
import jax
import jax.numpy as jnp
from jax.experimental import pallas as pl
from jax.experimental.pallas import tpu as pltpu

_BM = 400


def _gcn_kernel(a_ref, ax_ref, w0t_ref, b0_ref, a0_ref, w1t_ref, b1_ref,
                a1_ref, out_ref, t_ref):
    @pl.when(pl.program_id(0) == 0)
    def _compute_t():
        x = jnp.dot(ax_ref[...], w0t_ref[...], preferred_element_type=jnp.float32)
        x = x + b0_ref[...]
        a0 = a0_ref[0, 0]
        x = jnp.where(x >= 0, x, a0 * x)
        t = jnp.dot(x, w1t_ref[...], preferred_element_type=jnp.float32)
        t_ref[...] = t + b1_ref[...]

    acc = jnp.dot(a_ref[...], t_ref[...], preferred_element_type=jnp.float32)
    a1 = a1_ref[0, 0]
    out_ref[...] = jnp.where(acc >= 0, acc, a1 * acc)


def kernel(A, AX, W0, b0, a0, W1, b1, a1):
    n, d = AX.shape
    h = W0.shape[0]

    out = pl.pallas_call(
        _gcn_kernel,
        grid=(n // _BM,),
        in_specs=[
            pl.BlockSpec((_BM, n), lambda i: (i, 0)),
            pl.BlockSpec((n, d), lambda i: (0, 0)),
            pl.BlockSpec((d, h), lambda i: (0, 0)),
            pl.BlockSpec((1, h), lambda i: (0, 0)),
            pl.BlockSpec(memory_space=pltpu.SMEM),
            pl.BlockSpec((h, h), lambda i: (0, 0)),
            pl.BlockSpec((1, h), lambda i: (0, 0)),
            pl.BlockSpec(memory_space=pltpu.SMEM),
        ],
        out_specs=pl.BlockSpec((_BM, h), lambda i: (i, 0)),
        out_shape=jax.ShapeDtypeStruct((n, h), jnp.float32),
        scratch_shapes=[pltpu.VMEM((n, h), jnp.float32)],
        compiler_params=pltpu.CompilerParams(
            dimension_semantics=("arbitrary",),
        ),
    )(A, AX, W0.T, b0.reshape(1, h), a0.reshape(1, 1),
      W1.T, b1.reshape(1, h), a1.reshape(1, 1))

    return out[None, :, :]


# speedup vs baseline: 1.0666x; 1.0068x over previous
import jax
import jax.numpy as jnp
from jax.experimental import pallas as pl
from jax.experimental.pallas import tpu as pltpu

_BM = 200


def _gcn_kernel(a_ref, ax_ref, w0t_ref, b0_ref, a0_ref, w1t_ref, b1_ref,
                a1_ref, out_ref, t_ref):
    @pl.when(pl.program_id(0) == 0)
    def _compute_t():
        x = jnp.dot(ax_ref[...], w0t_ref[...], preferred_element_type=jnp.float32)
        x = x + b0_ref[...]
        a0 = a0_ref[0, 0]
        x = jnp.where(x >= 0, x, a0 * x)
        t = jnp.dot(x, w1t_ref[...], preferred_element_type=jnp.float32)
        t_ref[...] = t + b1_ref[...]

    acc = jnp.dot(a_ref[...], t_ref[...], preferred_element_type=jnp.float32)
    a1 = a1_ref[0, 0]
    out_ref[...] = jnp.where(acc >= 0, acc, a1 * acc)


def kernel(A, AX, W0, b0, a0, W1, b1, a1):
    n, d = AX.shape
    h = W0.shape[0]

    out = pl.pallas_call(
        _gcn_kernel,
        grid=(n // _BM,),
        in_specs=[
            pl.BlockSpec((_BM, n), lambda i: (i, 0)),
            pl.BlockSpec((n, d), lambda i: (0, 0)),
            pl.BlockSpec((d, h), lambda i: (0, 0)),
            pl.BlockSpec((1, h), lambda i: (0, 0)),
            pl.BlockSpec(memory_space=pltpu.SMEM),
            pl.BlockSpec((h, h), lambda i: (0, 0)),
            pl.BlockSpec((1, h), lambda i: (0, 0)),
            pl.BlockSpec(memory_space=pltpu.SMEM),
        ],
        out_specs=pl.BlockSpec((_BM, h), lambda i: (i, 0)),
        out_shape=jax.ShapeDtypeStruct((n, h), jnp.float32),
        scratch_shapes=[pltpu.VMEM((n, h), jnp.float32)],
        compiler_params=pltpu.CompilerParams(
            dimension_semantics=("arbitrary",),
        ),
    )(A, AX, W0.T, b0.reshape(1, h), a0.reshape(1, 1),
      W1.T, b1.reshape(1, h), a1.reshape(1, 1))

    return out[None, :, :]
